# Initial kernel scaffold; baseline (speedup 1.0000x reference)
import jax
import jax.numpy as jnp
from jax.experimental import pallas as pl


def kernel(feature, frame_level_feature, W1, b1, g1, be1, W2, b2, g2, be2, W3, b3, g3, be3):
    raise NotImplementedError("write your pallas kernel here")


# speedup vs baseline: 1.3756x; 1.3756x over previous
import functools

import jax
import jax.numpy as jnp
from jax import lax
from jax.experimental import pallas as pl
from jax.experimental.pallas import tpu as pltpu
from jax.experimental.pallas import tpu_sc as plsc

GROUPS = 32
EPS = 1e-5


_IT = 512
_JT = 1024


def _nearest_idx_kernel(L, c_ref, out_ref):
    base = pl.program_id(0) * _IT
    i_row = lax.broadcasted_iota(jnp.int32, (_JT, _IT), 1) + base
    rm = None
    ra = None
    for jt in range(L // _JT):
        c_t = c_ref[pl.ds(jt * _JT, _JT), :]
        j_col = lax.broadcasted_iota(jnp.int32, (_JT, _IT), 0) + jt * _JT
        d = jnp.abs(c_t - i_row)
        m = jnp.min(d, axis=0, keepdims=True)
        a = jnp.min(jnp.where(d == m, j_col, jnp.int32(2 * L)),
                    axis=0, keepdims=True)
        if rm is None:
            rm, ra = m, a
        else:
            upd = m < rm
            ra = jnp.where(upd, a, ra)
            rm = jnp.where(upd, m, rm)
    out_ref[0, 0, :] = ra[0]


def _nearest_idx(cdf_int, T):
    L = cdf_int.shape[0]
    out = pl.pallas_call(
        functools.partial(_nearest_idx_kernel, L),
        grid=(T // _IT,),
        in_specs=[pl.BlockSpec((L, 1), lambda i: (0, 0))],
        out_specs=pl.BlockSpec((1, 1, _IT), lambda i: (i, 0, 0)),
        out_shape=jax.ShapeDtypeStruct((T // _IT, 1, _IT), jnp.int32),
    )(cdf_int.reshape(L, 1))
    return out.reshape(T)


_NW = 32


def _make_sc_gather(R, L, T):
    rows_per = R // _NW
    mesh = plsc.VectorSubcoreMesh(core_axis_name="c", subcore_axis_name="s")

    @functools.partial(
        pl.kernel,
        mesh=mesh,
        out_type=jax.ShapeDtypeStruct((R, T), jnp.float32),
        scratch_types=[
            pltpu.VMEM((T,), jnp.int32),
            pltpu.VMEM((L,), jnp.float32),
            pltpu.VMEM((T,), jnp.float32),
        ],
        compiler_params=pltpu.CompilerParams(needs_layout_passes=False),
    )
    def gath(tab_hbm, idx_hbm, out_hbm, idx_v, row_v, out_v):
        wid = lax.axis_index("s") * 2 + lax.axis_index("c")

        pltpu.sync_copy(idx_hbm, idx_v)

        def row_body(r, carry):
            row = wid * rows_per + r
            pltpu.sync_copy(tab_hbm.at[row], row_v)

            def chunk(k, c2):
                iv = idx_v[pl.ds(k * 16, 16)]
                out_v[pl.ds(k * 16, 16)] = plsc.load_gather(row_v, [iv])
                return c2

            lax.fori_loop(0, T // 16, chunk, 0, unroll=8)
            pltpu.sync_copy(out_v, out_hbm.at[row])
            return carry

        lax.fori_loop(0, rows_per, row_body, 0)

    return gath


_TT = 512


def _group_masks(P, cg, dtype=jnp.float32):
    g = lax.broadcasted_iota(jnp.int32, (GROUPS, P), 0)
    c = lax.broadcasted_iota(jnp.int32, (GROUPS, P), 1)
    m = (c // cg == g).astype(dtype)
    gT = lax.broadcasted_iota(jnp.int32, (P, GROUPS), 1)
    cT = lax.broadcasted_iota(jnp.int32, (P, GROUPS), 0)
    mT = (cT // cg == gT).astype(dtype)
    return m, mT


def _stats_kernel(T, x_ref, w1_ref, b1_ref, w2_ref, b2_ref, stats_ref, gx, sx):
    t = pl.program_id(1)
    nt = pl.num_programs(1)
    x = x_ref[0]

    @pl.when(t == 0)
    def _init():
        gx[...] = jnp.zeros_like(gx)
        sx[...] = jnp.zeros_like(sx)

    gx[...] += lax.dot_general(x, x, (((1,), (1,)), ((), ())),
                               preferred_element_type=jnp.float32)
    sx[...] += jnp.sum(x, axis=1, keepdims=True)

    @pl.when(t == nt - 1)
    def _finish():
        Gx = gx[...]
        sxv = sx[...]

        def layer_stats(W, b, cg):
            P = W.shape[0]
            WG = jnp.dot(W, Gx, preferred_element_type=jnp.float32)
            q = jnp.sum(WG * W, axis=1, keepdims=True)
            u = jnp.dot(W, sxv, preferred_element_type=jnp.float32)
            sum_y = u + T * b
            sum_y2 = q + 2.0 * b * u + T * (b * b)
            mask, _ = _group_masks(P, cg)
            Sg = jnp.dot(mask, sum_y, preferred_element_type=jnp.float32)
            Qg = jnp.dot(mask, sum_y2, preferred_element_type=jnp.float32)
            n = cg * T
            mean = Sg / n
            var = Qg / n - mean * mean
            return mean, lax.rsqrt(var + EPS)

        m1, i1 = layer_stats(w1_ref[...], b1_ref[...], 256 // GROUPS)
        m2, i2 = layer_stats(w2_ref[...], b2_ref[...], 512 // GROUPS)
        stats_ref[0, 0:32] = m1
        stats_ref[0, 32:64] = i1
        stats_ref[0, 64:96] = m2
        stats_ref[0, 96:128] = i2


def _blockb_kernel(x_ref, smp_ref, stats_ref, w1_ref, b1_ref, g1_ref, be1_ref,
                   w2_ref, b2_ref, g2_ref, be2_ref, w3_ref, b3_ref,
                   feat_ref, y3_ref, s3_ref):
    t = pl.program_id(1)
    x = x_ref[0]
    smp = smp_ref[0]
    st = stats_ref[0]

    _, mT8 = _group_masks(256, 8)
    _, mT16 = _group_masks(512, 16)
    mask8, _ = _group_masks(256, 8)

    y1 = jnp.dot(w1_ref[...], x, preferred_element_type=jnp.float32) + b1_ref[...]
    m1e = jnp.dot(mT8, st[0:32], preferred_element_type=jnp.float32)
    i1e = jnp.dot(mT8, st[32:64], preferred_element_type=jnp.float32)
    h1 = jnp.maximum((y1 - m1e) * i1e * g1_ref[...] + be1_ref[...], 0.0)

    y2 = jnp.dot(w2_ref[...], x, preferred_element_type=jnp.float32) + b2_ref[...]
    m2e = jnp.dot(mT16, st[64:96], preferred_element_type=jnp.float32)
    i2e = jnp.dot(mT16, st[96:128], preferred_element_type=jnp.float32)
    h2 = jnp.maximum((y2 - m2e) * i2e * g2_ref[...] + be2_ref[...], 0.0)
    feat_ref[0] = h2

    y3 = (jnp.dot(w3_ref[:, 0:256], smp, preferred_element_type=jnp.float32)
          + jnp.dot(w3_ref[:, 256:768], h2, preferred_element_type=jnp.float32)
          + jnp.dot(w3_ref[:, 768:1024], h1, preferred_element_type=jnp.float32)
          + b3_ref[...])
    y3_ref[0] = y3

    ssum = jnp.dot(mask8, jnp.sum(y3, axis=1, keepdims=True),
                   preferred_element_type=jnp.float32)
    ssq = jnp.dot(mask8, jnp.sum(y3 * y3, axis=1, keepdims=True),
                  preferred_element_type=jnp.float32)

    @pl.when(t == 0)
    def _init():
        s3_ref[...] = jnp.zeros_like(s3_ref)

    s3_ref[0, 0:32] += ssum
    s3_ref[0, 32:64] += ssq


def _normc_kernel(N3, y3_ref, s3_ref, g3_ref, be3_ref, out_ref):
    y = y3_ref[0]
    st = s3_ref[0]
    m3 = st[0:32] / N3
    v3 = st[32:64] / N3 - m3 * m3
    i3 = lax.rsqrt(v3 + EPS)
    _, mT8 = _group_masks(256, 8)
    me = jnp.dot(mT8, m3, preferred_element_type=jnp.float32)
    ie = jnp.dot(mT8, i3, preferred_element_type=jnp.float32)
    out_ref[0] = jnp.maximum((y - me) * ie * g3_ref[...] + be3_ref[...], 0.0)


def kernel(feature, frame_level_feature, W1, b1, g1, be1, W2, b2, g2, be2,
           W3, b3, g3, be3):
    B, C, T = feature.shape
    L = frame_level_feature.shape[2]
    P1 = W1.shape[0]
    P2 = W2.shape[0]
    nt = T // _TT

    mean_values = jnp.mean(frame_level_feature, axis=1)[0]
    mean_values = mean_values / jnp.sum(mean_values)
    cdf_values = jnp.cumsum(mean_values)
    cdf_int = (lax.stop_gradient(cdf_values) * T).astype(jnp.int32)
    cdf_int = jnp.minimum(cdf_int, T - 1)

    idxs = _nearest_idx(cdf_int, T)

    tab = frame_level_feature.reshape(B * C, L)
    sampled = _make_sc_gather(B * C, L, T)(tab, idxs)
    sampled = sampled.reshape(B, C, T)

    b1c = b1.reshape(P1, 1)
    b2c = b2.reshape(P2, 1)
    b3c = b3.reshape(P1, 1)
    stats = pl.pallas_call(
        functools.partial(_stats_kernel, float(T)),
        grid=(B, nt),
        in_specs=[
            pl.BlockSpec((1, C, _TT), lambda b, t: (b, 0, t)),
            pl.BlockSpec((P1, C), lambda b, t: (0, 0)),
            pl.BlockSpec((P1, 1), lambda b, t: (0, 0)),
            pl.BlockSpec((P2, C), lambda b, t: (0, 0)),
            pl.BlockSpec((P2, 1), lambda b, t: (0, 0)),
        ],
        out_specs=pl.BlockSpec((1, 128, 1), lambda b, t: (b, 0, 0)),
        out_shape=jax.ShapeDtypeStruct((B, 128, 1), jnp.float32),
        scratch_shapes=[
            pltpu.VMEM((C, C), jnp.float32),
            pltpu.VMEM((C, 1), jnp.float32),
        ],
    )(feature, W1, b1c, W2, b2c)

    feat, y3raw, s3 = pl.pallas_call(
        _blockb_kernel,
        grid=(B, nt),
        in_specs=[
            pl.BlockSpec((1, C, _TT), lambda b, t: (b, 0, t)),
            pl.BlockSpec((1, C, _TT), lambda b, t: (b, 0, t)),
            pl.BlockSpec((1, 128, 1), lambda b, t: (b, 0, 0)),
            pl.BlockSpec((P1, C), lambda b, t: (0, 0)),
            pl.BlockSpec((P1, 1), lambda b, t: (0, 0)),
            pl.BlockSpec((P1, 1), lambda b, t: (0, 0)),
            pl.BlockSpec((P1, 1), lambda b, t: (0, 0)),
            pl.BlockSpec((P2, C), lambda b, t: (0, 0)),
            pl.BlockSpec((P2, 1), lambda b, t: (0, 0)),
            pl.BlockSpec((P2, 1), lambda b, t: (0, 0)),
            pl.BlockSpec((P2, 1), lambda b, t: (0, 0)),
            pl.BlockSpec((P1, 4 * P1), lambda b, t: (0, 0)),
            pl.BlockSpec((P1, 1), lambda b, t: (0, 0)),
        ],
        out_specs=[
            pl.BlockSpec((1, P2, _TT), lambda b, t: (b, 0, t)),
            pl.BlockSpec((1, P1, _TT), lambda b, t: (b, 0, t)),
            pl.BlockSpec((1, 64, 1), lambda b, t: (b, 0, 0)),
        ],
        out_shape=[
            jax.ShapeDtypeStruct((B, P2, T), jnp.float32),
            jax.ShapeDtypeStruct((B, P1, T), jnp.float32),
            jax.ShapeDtypeStruct((B, 64, 1), jnp.float32),
        ],
    )(feature, sampled, stats, W1, b1c, g1.reshape(P1, 1), be1.reshape(P1, 1),
      W2, b2c, g2.reshape(P2, 1), be2.reshape(P2, 1), W3, b3c)

    mixed = pl.pallas_call(
        functools.partial(_normc_kernel, float((P1 // GROUPS) * T)),
        grid=(B, nt),
        in_specs=[
            pl.BlockSpec((1, P1, _TT), lambda b, t: (b, 0, t)),
            pl.BlockSpec((1, 64, 1), lambda b, t: (b, 0, 0)),
            pl.BlockSpec((P1, 1), lambda b, t: (0, 0)),
            pl.BlockSpec((P1, 1), lambda b, t: (0, 0)),
        ],
        out_specs=pl.BlockSpec((1, P1, _TT), lambda b, t: (b, 0, t)),
        out_shape=jax.ShapeDtypeStruct((B, P1, T), jnp.float32),
    )(y3raw, s3, g3.reshape(P1, 1), be3.reshape(P1, 1))

    return (mixed, feat)


# speedup vs baseline: 1.4310x; 1.0403x over previous
import functools

import jax
import jax.numpy as jnp
from jax import lax
from jax.experimental import pallas as pl
from jax.experimental.pallas import tpu as pltpu
from jax.experimental.pallas import tpu_sc as plsc

GROUPS = 32
EPS = 1e-5


_IT = 512
_JT = 1024


def _nearest_idx_kernel(L, c_ref, out_ref):
    base = pl.program_id(0) * _IT
    i_row = lax.broadcasted_iota(jnp.int32, (_JT, _IT), 1) + base
    rm = None
    ra = None
    for jt in range(L // _JT):
        c_t = c_ref[pl.ds(jt * _JT, _JT), :]
        j_col = lax.broadcasted_iota(jnp.int32, (_JT, _IT), 0) + jt * _JT
        d = jnp.abs(c_t - i_row)
        m = jnp.min(d, axis=0, keepdims=True)
        a = jnp.min(jnp.where(d == m, j_col, jnp.int32(2 * L)),
                    axis=0, keepdims=True)
        if rm is None:
            rm, ra = m, a
        else:
            upd = m < rm
            ra = jnp.where(upd, a, ra)
            rm = jnp.where(upd, m, rm)
    out_ref[0, 0, :] = ra[0]


def _nearest_idx(cdf_int, T):
    L = cdf_int.shape[0]
    out = pl.pallas_call(
        functools.partial(_nearest_idx_kernel, L),
        grid=(T // _IT,),
        in_specs=[pl.BlockSpec((L, 1), lambda i: (0, 0))],
        out_specs=pl.BlockSpec((1, 1, _IT), lambda i: (i, 0, 0)),
        out_shape=jax.ShapeDtypeStruct((T // _IT, 1, _IT), jnp.int32),
    )(cdf_int.reshape(L, 1))
    return out.reshape(T)


_NW = 32


_QR = 4


def _make_sc_gather(R, L, T):
    quads = R // _QR
    quads_per = quads // _NW
    mesh = plsc.VectorSubcoreMesh(core_axis_name="c", subcore_axis_name="s")

    @functools.partial(
        pl.kernel,
        mesh=mesh,
        out_type=jax.ShapeDtypeStruct((quads, _QR * T), jnp.float32),
        scratch_types=[
            pltpu.VMEM((T,), jnp.int32),
            pltpu.VMEM((_QR * L,), jnp.float32),
            pltpu.VMEM((_QR * L,), jnp.float32),
            pltpu.VMEM((_QR * T,), jnp.float32),
            pltpu.VMEM((_QR * T,), jnp.float32),
            pltpu.SemaphoreType.DMA,
            pltpu.SemaphoreType.DMA,
            pltpu.SemaphoreType.DMA,
            pltpu.SemaphoreType.DMA,
        ],
        compiler_params=pltpu.CompilerParams(needs_layout_passes=False),
    )
    def gath(tab_hbm, idx_hbm, out_hbm, idx_v, rvA, rvB, ovA, ovB,
             isA, isB, osA, osB):
        wid = lax.axis_index("s") * 2 + lax.axis_index("c")
        base = wid * quads_per

        pltpu.sync_copy(idx_hbm, idx_v)
        pltpu.async_copy(tab_hbm.at[base], rvA, isA)
        pltpu.async_copy(tab_hbm.at[base + 1], rvB, isB)

        bufs = ((rvA, ovA, isA, osA), (rvB, ovB, isB, osB))

        def outer(g, carry):
            for b, (rv, ov, isem, osem) in enumerate(bufs):
                q = 2 * g + b
                pltpu.make_async_copy(tab_hbm.at[base], rv, isem).wait()
                @pl.when(q >= 2)
                def _drain():
                    pltpu.make_async_copy(ov, out_hbm.at[base], osem).wait()

                def chunk(k, c2):
                    iv = idx_v[pl.ds(k * 16, 16)]
                    for r in range(_QR):
                        ov[pl.ds(r * T + k * 16, 16)] = plsc.load_gather(
                            rv, [iv + r * L])
                    return c2

                lax.fori_loop(0, T // 16, chunk, 0, unroll=8)
                pltpu.async_copy(ov, out_hbm.at[base + q], osem)
                nxt = base + jnp.minimum(q + 2, quads_per - 1)
                pltpu.async_copy(tab_hbm.at[nxt], rv, isem)
            return carry

        lax.fori_loop(0, quads_per // 2, outer, 0)
        for rv, ov, isem, osem in bufs:
            pltpu.make_async_copy(tab_hbm.at[base], rv, isem).wait()
            pltpu.make_async_copy(ov, out_hbm.at[base], osem).wait()

    return gath


_TT = 512


def _group_masks(P, cg, dtype=jnp.float32):
    g = lax.broadcasted_iota(jnp.int32, (GROUPS, P), 0)
    c = lax.broadcasted_iota(jnp.int32, (GROUPS, P), 1)
    m = (c // cg == g).astype(dtype)
    gT = lax.broadcasted_iota(jnp.int32, (P, GROUPS), 1)
    cT = lax.broadcasted_iota(jnp.int32, (P, GROUPS), 0)
    mT = (cT // cg == gT).astype(dtype)
    return m, mT


def _stats_kernel(T, x_ref, w1_ref, b1_ref, w2_ref, b2_ref, stats_ref, gx, sx):
    t = pl.program_id(1)
    nt = pl.num_programs(1)
    x = x_ref[0]

    @pl.when(t == 0)
    def _init():
        gx[...] = jnp.zeros_like(gx)
        sx[...] = jnp.zeros_like(sx)

    gx[...] += lax.dot_general(x, x, (((1,), (1,)), ((), ())),
                               preferred_element_type=jnp.float32)
    sx[...] += jnp.sum(x, axis=1, keepdims=True)

    @pl.when(t == nt - 1)
    def _finish():
        Gx = gx[...]
        sxv = sx[...]

        def layer_stats(W, b, cg):
            P = W.shape[0]
            WG = jnp.dot(W, Gx, preferred_element_type=jnp.float32)
            q = jnp.sum(WG * W, axis=1, keepdims=True)
            u = jnp.dot(W, sxv, preferred_element_type=jnp.float32)
            sum_y = u + T * b
            sum_y2 = q + 2.0 * b * u + T * (b * b)
            mask, _ = _group_masks(P, cg)
            Sg = jnp.dot(mask, sum_y, preferred_element_type=jnp.float32)
            Qg = jnp.dot(mask, sum_y2, preferred_element_type=jnp.float32)
            n = cg * T
            mean = Sg / n
            var = Qg / n - mean * mean
            return mean, lax.rsqrt(var + EPS)

        m1, i1 = layer_stats(w1_ref[...], b1_ref[...], 256 // GROUPS)
        m2, i2 = layer_stats(w2_ref[...], b2_ref[...], 512 // GROUPS)
        stats_ref[0, 0:32] = m1
        stats_ref[0, 32:64] = i1
        stats_ref[0, 64:96] = m2
        stats_ref[0, 96:128] = i2


def _blockb_kernel(x_ref, smp_ref, stats_ref, w1_ref, b1_ref, g1_ref, be1_ref,
                   w2_ref, b2_ref, g2_ref, be2_ref, w3_ref, b3_ref,
                   feat_ref, y3_ref, s3_ref):
    t = pl.program_id(1)
    x = x_ref[0]
    smp = smp_ref[0]
    st = stats_ref[0]

    _, mT8 = _group_masks(256, 8)
    _, mT16 = _group_masks(512, 16)
    mask8, _ = _group_masks(256, 8)

    y1 = jnp.dot(w1_ref[...], x, preferred_element_type=jnp.float32) + b1_ref[...]
    m1e = jnp.dot(mT8, st[0:32], preferred_element_type=jnp.float32)
    i1e = jnp.dot(mT8, st[32:64], preferred_element_type=jnp.float32)
    h1 = jnp.maximum((y1 - m1e) * i1e * g1_ref[...] + be1_ref[...], 0.0)

    y2 = jnp.dot(w2_ref[...], x, preferred_element_type=jnp.float32) + b2_ref[...]
    m2e = jnp.dot(mT16, st[64:96], preferred_element_type=jnp.float32)
    i2e = jnp.dot(mT16, st[96:128], preferred_element_type=jnp.float32)
    h2 = jnp.maximum((y2 - m2e) * i2e * g2_ref[...] + be2_ref[...], 0.0)
    feat_ref[0] = h2

    y3 = (jnp.dot(w3_ref[:, 0:256], smp, preferred_element_type=jnp.float32)
          + jnp.dot(w3_ref[:, 256:768], h2, preferred_element_type=jnp.float32)
          + jnp.dot(w3_ref[:, 768:1024], h1, preferred_element_type=jnp.float32)
          + b3_ref[...])
    y3_ref[0] = y3

    ssum = jnp.dot(mask8, jnp.sum(y3, axis=1, keepdims=True),
                   preferred_element_type=jnp.float32)
    ssq = jnp.dot(mask8, jnp.sum(y3 * y3, axis=1, keepdims=True),
                  preferred_element_type=jnp.float32)

    @pl.when(t == 0)
    def _init():
        s3_ref[...] = jnp.zeros_like(s3_ref)

    s3_ref[0, 0:32] += ssum
    s3_ref[0, 32:64] += ssq


def _normc_kernel(N3, y3_ref, s3_ref, g3_ref, be3_ref, out_ref):
    y = y3_ref[0]
    st = s3_ref[0]
    m3 = st[0:32] / N3
    v3 = st[32:64] / N3 - m3 * m3
    i3 = lax.rsqrt(v3 + EPS)
    _, mT8 = _group_masks(256, 8)
    me = jnp.dot(mT8, m3, preferred_element_type=jnp.float32)
    ie = jnp.dot(mT8, i3, preferred_element_type=jnp.float32)
    out_ref[0] = jnp.maximum((y - me) * ie * g3_ref[...] + be3_ref[...], 0.0)


def kernel(feature, frame_level_feature, W1, b1, g1, be1, W2, b2, g2, be2,
           W3, b3, g3, be3):
    B, C, T = feature.shape
    L = frame_level_feature.shape[2]
    P1 = W1.shape[0]
    P2 = W2.shape[0]
    nt = T // _TT

    mean_values = jnp.mean(frame_level_feature, axis=1)[0]
    mean_values = mean_values / jnp.sum(mean_values)
    cdf_values = jnp.cumsum(mean_values)
    cdf_int = (lax.stop_gradient(cdf_values) * T).astype(jnp.int32)
    cdf_int = jnp.minimum(cdf_int, T - 1)

    idxs = _nearest_idx(cdf_int, T)

    tab = frame_level_feature.reshape(B * C // _QR, _QR * L)
    sampled = _make_sc_gather(B * C, L, T)(tab, idxs)
    sampled = sampled.reshape(B, C, T)

    b1c = b1.reshape(P1, 1)
    b2c = b2.reshape(P2, 1)
    b3c = b3.reshape(P1, 1)
    stats = pl.pallas_call(
        functools.partial(_stats_kernel, float(T)),
        grid=(B, nt),
        in_specs=[
            pl.BlockSpec((1, C, _TT), lambda b, t: (b, 0, t)),
            pl.BlockSpec((P1, C), lambda b, t: (0, 0)),
            pl.BlockSpec((P1, 1), lambda b, t: (0, 0)),
            pl.BlockSpec((P2, C), lambda b, t: (0, 0)),
            pl.BlockSpec((P2, 1), lambda b, t: (0, 0)),
        ],
        out_specs=pl.BlockSpec((1, 128, 1), lambda b, t: (b, 0, 0)),
        out_shape=jax.ShapeDtypeStruct((B, 128, 1), jnp.float32),
        scratch_shapes=[
            pltpu.VMEM((C, C), jnp.float32),
            pltpu.VMEM((C, 1), jnp.float32),
        ],
    )(feature, W1, b1c, W2, b2c)

    feat, y3raw, s3 = pl.pallas_call(
        _blockb_kernel,
        grid=(B, nt),
        in_specs=[
            pl.BlockSpec((1, C, _TT), lambda b, t: (b, 0, t)),
            pl.BlockSpec((1, C, _TT), lambda b, t: (b, 0, t)),
            pl.BlockSpec((1, 128, 1), lambda b, t: (b, 0, 0)),
            pl.BlockSpec((P1, C), lambda b, t: (0, 0)),
            pl.BlockSpec((P1, 1), lambda b, t: (0, 0)),
            pl.BlockSpec((P1, 1), lambda b, t: (0, 0)),
            pl.BlockSpec((P1, 1), lambda b, t: (0, 0)),
            pl.BlockSpec((P2, C), lambda b, t: (0, 0)),
            pl.BlockSpec((P2, 1), lambda b, t: (0, 0)),
            pl.BlockSpec((P2, 1), lambda b, t: (0, 0)),
            pl.BlockSpec((P2, 1), lambda b, t: (0, 0)),
            pl.BlockSpec((P1, 4 * P1), lambda b, t: (0, 0)),
            pl.BlockSpec((P1, 1), lambda b, t: (0, 0)),
        ],
        out_specs=[
            pl.BlockSpec((1, P2, _TT), lambda b, t: (b, 0, t)),
            pl.BlockSpec((1, P1, _TT), lambda b, t: (b, 0, t)),
            pl.BlockSpec((1, 64, 1), lambda b, t: (b, 0, 0)),
        ],
        out_shape=[
            jax.ShapeDtypeStruct((B, P2, T), jnp.float32),
            jax.ShapeDtypeStruct((B, P1, T), jnp.float32),
            jax.ShapeDtypeStruct((B, 64, 1), jnp.float32),
        ],
    )(feature, sampled, stats, W1, b1c, g1.reshape(P1, 1), be1.reshape(P1, 1),
      W2, b2c, g2.reshape(P2, 1), be2.reshape(P2, 1), W3, b3c)

    mixed = pl.pallas_call(
        functools.partial(_normc_kernel, float((P1 // GROUPS) * T)),
        grid=(B, nt),
        in_specs=[
            pl.BlockSpec((1, P1, _TT), lambda b, t: (b, 0, t)),
            pl.BlockSpec((1, 64, 1), lambda b, t: (b, 0, 0)),
            pl.BlockSpec((P1, 1), lambda b, t: (0, 0)),
            pl.BlockSpec((P1, 1), lambda b, t: (0, 0)),
        ],
        out_specs=pl.BlockSpec((1, P1, _TT), lambda b, t: (b, 0, t)),
        out_shape=jax.ShapeDtypeStruct((B, P1, T), jnp.float32),
    )(y3raw, s3, g3.reshape(P1, 1), be3.reshape(P1, 1))

    return (mixed, feat)


# speedup vs baseline: 2.1312x; 1.4893x over previous
import functools

import jax
import jax.numpy as jnp
from jax import lax
from jax.experimental import pallas as pl
from jax.experimental.pallas import tpu as pltpu
from jax.experimental.pallas import tpu_sc as plsc

GROUPS = 32
EPS = 1e-5


_IT = 512
_JT = 1024


def _nearest_idx_kernel(L, c_ref, out_ref):
    base = pl.program_id(0) * _IT
    i_row = lax.broadcasted_iota(jnp.int32, (_JT, _IT), 1) + base
    rm = None
    ra = None
    for jt in range(L // _JT):
        c_t = c_ref[pl.ds(jt * _JT, _JT), :]
        j_col = lax.broadcasted_iota(jnp.int32, (_JT, _IT), 0) + jt * _JT
        d = jnp.abs(c_t - i_row)
        m = jnp.min(d, axis=0, keepdims=True)
        a = jnp.min(jnp.where(d == m, j_col, jnp.int32(2 * L)),
                    axis=0, keepdims=True)
        if rm is None:
            rm, ra = m, a
        else:
            upd = m < rm
            ra = jnp.where(upd, a, ra)
            rm = jnp.where(upd, m, rm)
    out_ref[0, 0, :] = ra[0]


def _nearest_idx(cdf_int, T):
    L = cdf_int.shape[0]
    out = pl.pallas_call(
        functools.partial(_nearest_idx_kernel, L),
        grid=(T // _IT,),
        in_specs=[pl.BlockSpec((L, 1), lambda i: (0, 0))],
        out_specs=pl.BlockSpec((1, 1, _IT), lambda i: (i, 0, 0)),
        out_shape=jax.ShapeDtypeStruct((T // _IT, 1, _IT), jnp.int32),
    )(cdf_int.reshape(L, 1))
    return out.reshape(T)


_NW = 32


_QR = 4


def _make_sc_gather(B, C, L, T):
    QC = C // _QR
    quads = B * QC
    quads_per = quads // _NW
    mesh = plsc.VectorSubcoreMesh(core_axis_name="c", subcore_axis_name="s")

    @functools.partial(
        pl.kernel,
        mesh=mesh,
        out_type=jax.ShapeDtypeStruct((B, C, T), jnp.float32),
        scratch_types=[
            pltpu.VMEM((T,), jnp.int32),
            pltpu.VMEM((_QR, L), jnp.float32),
            pltpu.VMEM((_QR, L), jnp.float32),
            pltpu.VMEM((_QR, T), jnp.float32),
            pltpu.VMEM((_QR, T), jnp.float32),
            pltpu.SemaphoreType.DMA,
            pltpu.SemaphoreType.DMA,
            pltpu.SemaphoreType.DMA,
            pltpu.SemaphoreType.DMA,
        ],
        compiler_params=pltpu.CompilerParams(needs_layout_passes=False),
    )
    def gath(tab_hbm, idx_hbm, out_hbm, idx_v, rvA, rvB, ovA, ovB,
             isA, isB, osA, osB):
        wid = lax.axis_index("s") * 2 + lax.axis_index("c")
        base = wid * quads_per

        def hbm_in(q):
            return tab_hbm.at[q // QC, pl.ds((q % QC) * _QR, _QR)]

        def hbm_out(q):
            return out_hbm.at[q // QC, pl.ds((q % QC) * _QR, _QR)]

        pltpu.sync_copy(idx_hbm, idx_v)
        pltpu.async_copy(hbm_in(base), rvA, isA)
        pltpu.async_copy(hbm_in(base + 1), rvB, isB)

        bufs = ((rvA, ovA, isA, osA), (rvB, ovB, isB, osB))

        def outer(g, carry):
            for b, (rv, ov, isem, osem) in enumerate(bufs):
                q = base + 2 * g + b
                pltpu.make_async_copy(hbm_in(base), rv, isem).wait()
                @pl.when(2 * g + b >= 2)
                def _drain():
                    pltpu.make_async_copy(ov, hbm_out(base), osem).wait()

                def chunk(k, c2):
                    iv = idx_v[pl.ds(k * 16, 16)]
                    for r in range(_QR):
                        rsp = jnp.full((16,), r, jnp.int32)
                        ov[r, pl.ds(k * 16, 16)] = plsc.load_gather(
                            rv, [rsp, iv])
                    return c2

                lax.fori_loop(0, T // 16, chunk, 0, unroll=8)
                pltpu.async_copy(ov, hbm_out(q), osem)
                nxt = jnp.minimum(q + 2, base + quads_per - 1)
                pltpu.async_copy(hbm_in(nxt), rv, isem)
            return carry

        lax.fori_loop(0, quads_per // 2, outer, 0)
        for rv, ov, isem, osem in bufs:
            pltpu.make_async_copy(hbm_in(base), rv, isem).wait()
            pltpu.make_async_copy(ov, hbm_out(base), osem).wait()

    return gath


_TT = 1024


def _group_masks(P, cg, dtype=jnp.float32):
    g = lax.broadcasted_iota(jnp.int32, (GROUPS, P), 0)
    c = lax.broadcasted_iota(jnp.int32, (GROUPS, P), 1)
    m = (c // cg == g).astype(dtype)
    gT = lax.broadcasted_iota(jnp.int32, (P, GROUPS), 1)
    cT = lax.broadcasted_iota(jnp.int32, (P, GROUPS), 0)
    mT = (cT // cg == gT).astype(dtype)
    return m, mT


def _stats_kernel(T, x_ref, w1_ref, b1_ref, w2_ref, b2_ref, stats_ref, gx, sx):
    t = pl.program_id(1)
    nt = pl.num_programs(1)
    x = x_ref[0]

    @pl.when(t == 0)
    def _init():
        gx[...] = jnp.zeros_like(gx)
        sx[...] = jnp.zeros_like(sx)

    xb = x.astype(jnp.bfloat16)
    gx[...] += lax.dot_general(xb, xb, (((1,), (1,)), ((), ())),
                               preferred_element_type=jnp.float32)
    sx[...] += jnp.sum(x, axis=1, keepdims=True)

    @pl.when(t == nt - 1)
    def _finish():
        Gx = gx[...]
        sxv = sx[...]

        def layer_stats(W, b, cg):
            P = W.shape[0]
            WG = jnp.dot(W, Gx, preferred_element_type=jnp.float32)
            q = jnp.sum(WG * W, axis=1, keepdims=True)
            u = jnp.dot(W, sxv, preferred_element_type=jnp.float32)
            sum_y = u + T * b
            sum_y2 = q + 2.0 * b * u + T * (b * b)
            mask, _ = _group_masks(P, cg)
            Sg = jnp.dot(mask, sum_y, preferred_element_type=jnp.float32)
            Qg = jnp.dot(mask, sum_y2, preferred_element_type=jnp.float32)
            n = cg * T
            mean = Sg / n
            var = Qg / n - mean * mean
            return mean, lax.rsqrt(var + EPS)

        m1, i1 = layer_stats(w1_ref[...], b1_ref[...], 256 // GROUPS)
        m2, i2 = layer_stats(w2_ref[...], b2_ref[...], 512 // GROUPS)
        stats_ref[0, 0:32] = m1
        stats_ref[0, 32:64] = i1
        stats_ref[0, 64:96] = m2
        stats_ref[0, 96:128] = i2


def _blockb_kernel(x_ref, smp_ref, stats_ref, w1_ref, b1_ref, g1_ref, be1_ref,
                   w2_ref, b2_ref, g2_ref, be2_ref, w3_ref, b3_ref,
                   feat_ref, y3_ref, s3_ref):
    t = pl.program_id(1)
    x = x_ref[0]
    smp = smp_ref[0]
    st = stats_ref[0]

    _, mT8 = _group_masks(256, 8)
    _, mT16 = _group_masks(512, 16)
    mask8, _ = _group_masks(256, 8)

    y1 = jnp.dot(w1_ref[...], x, preferred_element_type=jnp.float32) + b1_ref[...]
    m1e = jnp.dot(mT8, st[0:32], preferred_element_type=jnp.float32)
    i1e = jnp.dot(mT8, st[32:64], preferred_element_type=jnp.float32)
    h1 = jnp.maximum((y1 - m1e) * i1e * g1_ref[...] + be1_ref[...], 0.0)

    y2 = jnp.dot(w2_ref[...], x, preferred_element_type=jnp.float32) + b2_ref[...]
    m2e = jnp.dot(mT16, st[64:96], preferred_element_type=jnp.float32)
    i2e = jnp.dot(mT16, st[96:128], preferred_element_type=jnp.float32)
    h2 = jnp.maximum((y2 - m2e) * i2e * g2_ref[...] + be2_ref[...], 0.0)
    feat_ref[0] = h2

    y3 = (jnp.dot(w3_ref[:, 0:256], smp, preferred_element_type=jnp.float32)
          + jnp.dot(w3_ref[:, 256:768], h2, preferred_element_type=jnp.float32)
          + jnp.dot(w3_ref[:, 768:1024], h1, preferred_element_type=jnp.float32)
          + b3_ref[...])
    y3_ref[0] = y3

    ssum = jnp.dot(mask8, jnp.sum(y3, axis=1, keepdims=True),
                   preferred_element_type=jnp.float32)
    ssq = jnp.dot(mask8, jnp.sum(y3 * y3, axis=1, keepdims=True),
                  preferred_element_type=jnp.float32)

    @pl.when(t == 0)
    def _init():
        s3_ref[...] = jnp.zeros_like(s3_ref)

    s3_ref[0, 0:32] += ssum
    s3_ref[0, 32:64] += ssq


def _normc_kernel(N3, y3_ref, s3_ref, g3_ref, be3_ref, out_ref):
    y = y3_ref[0]
    st = s3_ref[0]
    m3 = st[0:32] / N3
    v3 = st[32:64] / N3 - m3 * m3
    i3 = lax.rsqrt(v3 + EPS)
    _, mT8 = _group_masks(256, 8)
    me = jnp.dot(mT8, m3, preferred_element_type=jnp.float32)
    ie = jnp.dot(mT8, i3, preferred_element_type=jnp.float32)
    out_ref[0] = jnp.maximum((y - me) * ie * g3_ref[...] + be3_ref[...], 0.0)


def kernel(feature, frame_level_feature, W1, b1, g1, be1, W2, b2, g2, be2,
           W3, b3, g3, be3):
    B, C, T = feature.shape
    L = frame_level_feature.shape[2]
    P1 = W1.shape[0]
    P2 = W2.shape[0]
    nt = T // _TT

    mean_values = jnp.mean(frame_level_feature, axis=1)[0]
    mean_values = mean_values / jnp.sum(mean_values)
    cdf_values = jnp.cumsum(mean_values)
    cdf_int = (lax.stop_gradient(cdf_values) * T).astype(jnp.int32)
    cdf_int = jnp.minimum(cdf_int, T - 1)

    idxs = _nearest_idx(cdf_int, T)

    sampled = _make_sc_gather(B, C, L, T)(frame_level_feature, idxs)

    b1c = b1.reshape(P1, 1)
    b2c = b2.reshape(P2, 1)
    b3c = b3.reshape(P1, 1)
    stats = pl.pallas_call(
        functools.partial(_stats_kernel, float(T)),
        grid=(B, nt),
        in_specs=[
            pl.BlockSpec((1, C, _TT), lambda b, t: (b, 0, t)),
            pl.BlockSpec((P1, C), lambda b, t: (0, 0)),
            pl.BlockSpec((P1, 1), lambda b, t: (0, 0)),
            pl.BlockSpec((P2, C), lambda b, t: (0, 0)),
            pl.BlockSpec((P2, 1), lambda b, t: (0, 0)),
        ],
        out_specs=pl.BlockSpec((1, 128, 1), lambda b, t: (b, 0, 0)),
        out_shape=jax.ShapeDtypeStruct((B, 128, 1), jnp.float32),
        scratch_shapes=[
            pltpu.VMEM((C, C), jnp.float32),
            pltpu.VMEM((C, 1), jnp.float32),
        ],
    )(feature, W1, b1c, W2, b2c)

    feat, y3raw, s3 = pl.pallas_call(
        _blockb_kernel,
        grid=(B, nt),
        in_specs=[
            pl.BlockSpec((1, C, _TT), lambda b, t: (b, 0, t)),
            pl.BlockSpec((1, C, _TT), lambda b, t: (b, 0, t)),
            pl.BlockSpec((1, 128, 1), lambda b, t: (b, 0, 0)),
            pl.BlockSpec((P1, C), lambda b, t: (0, 0)),
            pl.BlockSpec((P1, 1), lambda b, t: (0, 0)),
            pl.BlockSpec((P1, 1), lambda b, t: (0, 0)),
            pl.BlockSpec((P1, 1), lambda b, t: (0, 0)),
            pl.BlockSpec((P2, C), lambda b, t: (0, 0)),
            pl.BlockSpec((P2, 1), lambda b, t: (0, 0)),
            pl.BlockSpec((P2, 1), lambda b, t: (0, 0)),
            pl.BlockSpec((P2, 1), lambda b, t: (0, 0)),
            pl.BlockSpec((P1, 4 * P1), lambda b, t: (0, 0)),
            pl.BlockSpec((P1, 1), lambda b, t: (0, 0)),
        ],
        out_specs=[
            pl.BlockSpec((1, P2, _TT), lambda b, t: (b, 0, t)),
            pl.BlockSpec((1, P1, _TT), lambda b, t: (b, 0, t)),
            pl.BlockSpec((1, 64, 1), lambda b, t: (b, 0, 0)),
        ],
        out_shape=[
            jax.ShapeDtypeStruct((B, P2, T), jnp.float32),
            jax.ShapeDtypeStruct((B, P1, T), jnp.float32),
            jax.ShapeDtypeStruct((B, 64, 1), jnp.float32),
        ],
    )(feature, sampled, stats, W1, b1c, g1.reshape(P1, 1), be1.reshape(P1, 1),
      W2, b2c, g2.reshape(P2, 1), be2.reshape(P2, 1), W3, b3c)

    mixed = pl.pallas_call(
        functools.partial(_normc_kernel, float((P1 // GROUPS) * T)),
        grid=(B, nt),
        in_specs=[
            pl.BlockSpec((1, P1, _TT), lambda b, t: (b, 0, t)),
            pl.BlockSpec((1, 64, 1), lambda b, t: (b, 0, 0)),
            pl.BlockSpec((P1, 1), lambda b, t: (0, 0)),
            pl.BlockSpec((P1, 1), lambda b, t: (0, 0)),
        ],
        out_specs=pl.BlockSpec((1, P1, _TT), lambda b, t: (b, 0, t)),
        out_shape=jax.ShapeDtypeStruct((B, P1, T), jnp.float32),
    )(y3raw, s3, g3.reshape(P1, 1), be3.reshape(P1, 1))

    return (mixed, feat)


# speedup vs baseline: 3.3264x; 1.5608x over previous
import functools

import jax
import jax.numpy as jnp
from jax import lax
from jax.experimental import pallas as pl
from jax.experimental.pallas import tpu as pltpu
from jax.experimental.pallas import tpu_sc as plsc

GROUPS = 32
EPS = 1e-5


_NW = 32


_QR = 4


def _make_sc_sample(B, C, L, T):
    QC = C // _QR
    quads = B * QC
    quads_per = quads // _NW
    NV = T // 16
    mesh = plsc.VectorSubcoreMesh(core_axis_name="c", subcore_axis_name="s")

    @functools.partial(
        pl.kernel,
        mesh=mesh,
        out_type=jax.ShapeDtypeStruct((B, C, T), jnp.float32),
        scratch_types=[
            pltpu.VMEM((L,), jnp.int32),
            pltpu.VMEM((T,), jnp.int32),
            pltpu.VMEM((T,), jnp.int32),
            pltpu.VMEM((_QR * L,), jnp.float32),
            pltpu.VMEM((_QR * L,), jnp.float32),
            pltpu.VMEM((_QR * T,), jnp.float32),
            pltpu.VMEM((_QR * T,), jnp.float32),
            pltpu.SemaphoreType.DMA,
            pltpu.SemaphoreType.DMA,
            pltpu.SemaphoreType.DMA,
            pltpu.SemaphoreType.DMA,
        ],
        compiler_params=pltpu.CompilerParams(needs_layout_passes=False),
    )
    def samp(tab_hbm, cdf_hbm, out_hbm, c_v, cnt_v, idx_v,
             rvA, rvB, ovA, ovB, isA, isB, osA, osB):
        wid = lax.axis_index("s") * 2 + lax.axis_index("c")
        base = wid * quads_per

        def row_src(q, r):
            return tab_hbm.at[q // QC, (q % QC) * _QR + r]

        def row_dst(q, r):
            return out_hbm.at[q // QC, (q % QC) * _QR + r]

        def start_in(q, rv, isem):
            for r in range(_QR):
                pltpu.async_copy(row_src(q, r), rv.at[pl.ds(r * L, L)], isem)

        def start_out(q, ov, osem):
            for r in range(_QR):
                pltpu.async_copy(ov.at[pl.ds(r * T, T)], row_dst(q, r), osem)

        def wait_in(rv, isem):
            for r in range(_QR):
                pltpu.make_async_copy(row_src(base, 0),
                                      rv.at[pl.ds(r * L, L)], isem).wait()

        def wait_out(ov, osem):
            for r in range(_QR):
                pltpu.make_async_copy(ov.at[pl.ds(r * T, T)],
                                      row_dst(base, 0), osem).wait()

        start_in(base, rvA, isA)
        start_in(base + 1, rvB, isB)
        pltpu.sync_copy(cdf_hbm, c_v)

        lane = lax.iota(jnp.int32, 16)

        @plsc.parallel_loop(0, NV, unroll=2)
        def _build_cnt(g):
            tv = g * 16 + lane
            cnt = jnp.zeros((16,), jnp.int32)
            step = L
            while step >= 1:
                cand = jnp.minimum(cnt + step, L)
                probe = plsc.load_gather(c_v, [cand - 1])
                cnt = jnp.where(probe <= tv, cand, cnt)
                step //= 2
            cnt_v[pl.ds(g * 16, 16)] = cnt

        @plsc.parallel_loop(0, NV, unroll=2)
        def _resolve(g):
            ii = g * 16 + lane
            k = cnt_v[pl.ds(g * 16, 16)]
            a = plsc.load_gather(c_v, [jnp.maximum(k - 1, 0)])
            bv = plsc.load_gather(c_v, [jnp.minimum(k, L - 1)])
            fa = plsc.load_gather(cnt_v, [jnp.maximum(a - 1, 0)])
            fa = jnp.where(a == 0, 0, fa)
            take_a = (k == L) | ((ii - a) <= (bv - ii))
            idx_v[pl.ds(g * 16, 16)] = jnp.where(
                k == 0, 0, jnp.where(take_a, fa, k))

        bufs = ((rvA, ovA, isA, osA), (rvB, ovB, isB, osB))

        def outer(g, carry):
            for b, (rv, ov, isem, osem) in enumerate(bufs):
                q = base + 2 * g + b
                wait_in(rv, isem)

                @pl.when(2 * g + b >= 2)
                def _drain():
                    wait_out(ov, osem)

                @plsc.parallel_loop(0, NV, unroll=8)
                def _chunk(k):
                    kk = k * 16
                    iv = idx_v[pl.ds(kk, 16)]
                    for r in range(_QR):
                        src = [iv + r * L] if r else [iv]
                        ov[pl.ds(r * T + kk, 16)] = plsc.load_gather(rv, src)

                start_out(q, ov, osem)
                nxt = jnp.minimum(q + 2, base + quads_per - 1)
                start_in(nxt, rv, isem)
            return carry

        lax.fori_loop(0, quads_per // 2, outer, 0)
        for rv, ov, isem, osem in bufs:
            wait_in(rv, isem)
            wait_out(ov, osem)

    return samp


_TT = 1024


def _group_masks(P, cg, dtype=jnp.float32):
    g = lax.broadcasted_iota(jnp.int32, (GROUPS, P), 0)
    c = lax.broadcasted_iota(jnp.int32, (GROUPS, P), 1)
    m = (c // cg == g).astype(dtype)
    gT = lax.broadcasted_iota(jnp.int32, (P, GROUPS), 1)
    cT = lax.broadcasted_iota(jnp.int32, (P, GROUPS), 0)
    mT = (cT // cg == gT).astype(dtype)
    return m, mT


def _stats_kernel(T, x_ref, w1_ref, b1_ref, w2_ref, b2_ref, stats_ref, gx, sx):
    t = pl.program_id(1)
    nt = pl.num_programs(1)
    x = x_ref[0]

    @pl.when(t == 0)
    def _init():
        gx[...] = jnp.zeros_like(gx)
        sx[...] = jnp.zeros_like(sx)

    xb = x.astype(jnp.bfloat16)
    gx[...] += lax.dot_general(xb, xb, (((1,), (1,)), ((), ())),
                               preferred_element_type=jnp.float32)
    sx[...] += jnp.sum(x, axis=1, keepdims=True)

    @pl.when(t == nt - 1)
    def _finish():
        Gx = gx[...]
        sxv = sx[...]

        def layer_stats(W, b, cg):
            P = W.shape[0]
            WG = jnp.dot(W, Gx, preferred_element_type=jnp.float32)
            q = jnp.sum(WG * W, axis=1, keepdims=True)
            u = jnp.dot(W, sxv, preferred_element_type=jnp.float32)
            sum_y = u + T * b
            sum_y2 = q + 2.0 * b * u + T * (b * b)
            mask, _ = _group_masks(P, cg)
            Sg = jnp.dot(mask, sum_y, preferred_element_type=jnp.float32)
            Qg = jnp.dot(mask, sum_y2, preferred_element_type=jnp.float32)
            n = cg * T
            mean = Sg / n
            var = Qg / n - mean * mean
            return mean, lax.rsqrt(var + EPS)

        m1, i1 = layer_stats(w1_ref[...], b1_ref[...], 256 // GROUPS)
        m2, i2 = layer_stats(w2_ref[...], b2_ref[...], 512 // GROUPS)
        stats_ref[0, 0:32] = m1
        stats_ref[0, 32:64] = i1
        stats_ref[0, 64:96] = m2
        stats_ref[0, 96:128] = i2


def _blockb_kernel(x_ref, smp_ref, stats_ref, w1_ref, b1_ref, g1_ref, be1_ref,
                   w2_ref, b2_ref, g2_ref, be2_ref, w3_ref, b3_ref,
                   feat_ref, y3_ref, s3_ref):
    t = pl.program_id(1)
    x = x_ref[0]
    smp = smp_ref[0]
    st = stats_ref[0]

    _, mT8 = _group_masks(256, 8)
    _, mT16 = _group_masks(512, 16)
    mask8, _ = _group_masks(256, 8)

    y1 = jnp.dot(w1_ref[...], x, preferred_element_type=jnp.float32) + b1_ref[...]
    m1e = jnp.dot(mT8, st[0:32], preferred_element_type=jnp.float32)
    i1e = jnp.dot(mT8, st[32:64], preferred_element_type=jnp.float32)
    h1 = jnp.maximum((y1 - m1e) * i1e * g1_ref[...] + be1_ref[...], 0.0)

    y2 = jnp.dot(w2_ref[...], x, preferred_element_type=jnp.float32) + b2_ref[...]
    m2e = jnp.dot(mT16, st[64:96], preferred_element_type=jnp.float32)
    i2e = jnp.dot(mT16, st[96:128], preferred_element_type=jnp.float32)
    h2 = jnp.maximum((y2 - m2e) * i2e * g2_ref[...] + be2_ref[...], 0.0)
    feat_ref[0] = h2

    y3 = (jnp.dot(w3_ref[:, 0:256], smp, preferred_element_type=jnp.float32)
          + jnp.dot(w3_ref[:, 256:768], h2, preferred_element_type=jnp.float32)
          + jnp.dot(w3_ref[:, 768:1024], h1, preferred_element_type=jnp.float32)
          + b3_ref[...])
    y3_ref[0] = y3.astype(jnp.bfloat16)

    ssum = jnp.dot(mask8, jnp.sum(y3, axis=1, keepdims=True),
                   preferred_element_type=jnp.float32)
    ssq = jnp.dot(mask8, jnp.sum(y3 * y3, axis=1, keepdims=True),
                  preferred_element_type=jnp.float32)

    @pl.when(t == 0)
    def _init():
        s3_ref[...] = jnp.zeros_like(s3_ref)

    s3_ref[0, 0:32] += ssum
    s3_ref[0, 32:64] += ssq


def _normc_kernel(N3, y3_ref, s3_ref, g3_ref, be3_ref, out_ref):
    y = y3_ref[0].astype(jnp.float32)
    st = s3_ref[0]
    m3 = st[0:32] / N3
    v3 = st[32:64] / N3 - m3 * m3
    i3 = lax.rsqrt(v3 + EPS)
    _, mT8 = _group_masks(256, 8)
    me = jnp.dot(mT8, m3, preferred_element_type=jnp.float32)
    ie = jnp.dot(mT8, i3, preferred_element_type=jnp.float32)
    out_ref[0] = jnp.maximum((y - me) * ie * g3_ref[...] + be3_ref[...], 0.0)


def kernel(feature, frame_level_feature, W1, b1, g1, be1, W2, b2, g2, be2,
           W3, b3, g3, be3):
    B, C, T = feature.shape
    L = frame_level_feature.shape[2]
    P1 = W1.shape[0]
    P2 = W2.shape[0]
    nt = T // _TT

    mean_values = jnp.mean(frame_level_feature[0:1], axis=1)[0]
    mean_values = mean_values / jnp.sum(mean_values)
    cdf_values = jnp.cumsum(mean_values)
    cdf_int = (lax.stop_gradient(cdf_values) * T).astype(jnp.int32)
    cdf_int = jnp.minimum(cdf_int, T - 1)

    sampled = _make_sc_sample(B, C, L, T)(frame_level_feature, cdf_int)

    b1c = b1.reshape(P1, 1)
    b2c = b2.reshape(P2, 1)
    b3c = b3.reshape(P1, 1)
    tts = 2048
    stats = pl.pallas_call(
        functools.partial(_stats_kernel, float(T)),
        grid=(B, T // tts),
        in_specs=[
            pl.BlockSpec((1, C, tts), lambda b, t: (b, 0, t)),
            pl.BlockSpec((P1, C), lambda b, t: (0, 0)),
            pl.BlockSpec((P1, 1), lambda b, t: (0, 0)),
            pl.BlockSpec((P2, C), lambda b, t: (0, 0)),
            pl.BlockSpec((P2, 1), lambda b, t: (0, 0)),
        ],
        out_specs=pl.BlockSpec((1, 128, 1), lambda b, t: (b, 0, 0)),
        out_shape=jax.ShapeDtypeStruct((B, 128, 1), jnp.float32),
        scratch_shapes=[
            pltpu.VMEM((C, C), jnp.float32),
            pltpu.VMEM((C, 1), jnp.float32),
        ],
    )(feature, W1, b1c, W2, b2c)

    feat, y3raw, s3 = pl.pallas_call(
        _blockb_kernel,
        grid=(B, nt),
        in_specs=[
            pl.BlockSpec((1, C, _TT), lambda b, t: (b, 0, t)),
            pl.BlockSpec((1, C, _TT), lambda b, t: (b, 0, t)),
            pl.BlockSpec((1, 128, 1), lambda b, t: (b, 0, 0)),
            pl.BlockSpec((P1, C), lambda b, t: (0, 0)),
            pl.BlockSpec((P1, 1), lambda b, t: (0, 0)),
            pl.BlockSpec((P1, 1), lambda b, t: (0, 0)),
            pl.BlockSpec((P1, 1), lambda b, t: (0, 0)),
            pl.BlockSpec((P2, C), lambda b, t: (0, 0)),
            pl.BlockSpec((P2, 1), lambda b, t: (0, 0)),
            pl.BlockSpec((P2, 1), lambda b, t: (0, 0)),
            pl.BlockSpec((P2, 1), lambda b, t: (0, 0)),
            pl.BlockSpec((P1, 4 * P1), lambda b, t: (0, 0)),
            pl.BlockSpec((P1, 1), lambda b, t: (0, 0)),
        ],
        out_specs=[
            pl.BlockSpec((1, P2, _TT), lambda b, t: (b, 0, t)),
            pl.BlockSpec((1, P1, _TT), lambda b, t: (b, 0, t)),
            pl.BlockSpec((1, 64, 1), lambda b, t: (b, 0, 0)),
        ],
        out_shape=[
            jax.ShapeDtypeStruct((B, P2, T), jnp.float32),
            jax.ShapeDtypeStruct((B, P1, T), jnp.bfloat16),
            jax.ShapeDtypeStruct((B, 64, 1), jnp.float32),
        ],
    )(feature, sampled, stats, W1, b1c, g1.reshape(P1, 1), be1.reshape(P1, 1),
      W2, b2c, g2.reshape(P2, 1), be2.reshape(P2, 1), W3, b3c)

    mixed = pl.pallas_call(
        functools.partial(_normc_kernel, float((P1 // GROUPS) * T)),
        grid=(B, nt),
        in_specs=[
            pl.BlockSpec((1, P1, _TT), lambda b, t: (b, 0, t)),
            pl.BlockSpec((1, 64, 1), lambda b, t: (b, 0, 0)),
            pl.BlockSpec((P1, 1), lambda b, t: (0, 0)),
            pl.BlockSpec((P1, 1), lambda b, t: (0, 0)),
        ],
        out_specs=pl.BlockSpec((1, P1, _TT), lambda b, t: (b, 0, t)),
        out_shape=jax.ShapeDtypeStruct((B, P1, T), jnp.float32),
    )(y3raw, s3, g3.reshape(P1, 1), be3.reshape(P1, 1))

    return (mixed, feat)


# speedup vs baseline: 3.7845x; 1.1377x over previous
import functools

import jax
import jax.numpy as jnp
from jax import lax
from jax.experimental import pallas as pl
from jax.experimental.pallas import tpu as pltpu
from jax.experimental.pallas import tpu_sc as plsc

GROUPS = 32
EPS = 1e-5


_NW = 32


_QR = 4


def _make_sc_sample(B, C, L, T):
    QC = C // _QR
    quads = B * QC
    quads_per = quads // _NW
    NV = T // 16
    mesh = plsc.VectorSubcoreMesh(core_axis_name="c", subcore_axis_name="s")

    @functools.partial(
        pl.kernel,
        mesh=mesh,
        out_type=jax.ShapeDtypeStruct((B, C, T), jnp.float32),
        scratch_types=[
            pltpu.VMEM((L,), jnp.int32),
            pltpu.VMEM((T,), jnp.int32),
            pltpu.VMEM((T,), jnp.int32),
            pltpu.VMEM((_QR * L,), jnp.float32),
            pltpu.VMEM((_QR * L,), jnp.float32),
            pltpu.VMEM((_QR * T,), jnp.float32),
            pltpu.VMEM((_QR * T,), jnp.float32),
            pltpu.SemaphoreType.DMA,
            pltpu.SemaphoreType.DMA,
            pltpu.SemaphoreType.DMA,
            pltpu.SemaphoreType.DMA,
        ],
        compiler_params=pltpu.CompilerParams(needs_layout_passes=False),
    )
    def samp(tab_hbm, cdf_hbm, out_hbm, c_v, cnt_v, idx_v,
             rvA, rvB, ovA, ovB, isA, isB, osA, osB):
        wid = lax.axis_index("s") * 2 + lax.axis_index("c")
        base = wid * quads_per

        def row_src(q, r):
            return tab_hbm.at[q // QC, (q % QC) * _QR + r]

        def row_dst(q, r):
            return out_hbm.at[q // QC, (q % QC) * _QR + r]

        def start_in(q, rv, isem):
            for r in range(_QR):
                pltpu.async_copy(row_src(q, r), rv.at[pl.ds(r * L, L)], isem)

        def start_out(q, ov, osem):
            for r in range(_QR):
                pltpu.async_copy(ov.at[pl.ds(r * T, T)], row_dst(q, r), osem)

        def wait_in(rv, isem):
            for r in range(_QR):
                pltpu.make_async_copy(row_src(base, 0),
                                      rv.at[pl.ds(r * L, L)], isem).wait()

        def wait_out(ov, osem):
            for r in range(_QR):
                pltpu.make_async_copy(ov.at[pl.ds(r * T, T)],
                                      row_dst(base, 0), osem).wait()

        start_in(base, rvA, isA)
        start_in(base + 1, rvB, isB)
        pltpu.sync_copy(cdf_hbm, c_v)

        lane = lax.iota(jnp.int32, 16)

        @plsc.parallel_loop(0, NV, unroll=2)
        def _build_cnt(g):
            tv = g * 16 + lane
            cnt = jnp.zeros((16,), jnp.int32)
            step = L
            while step >= 1:
                cand = jnp.minimum(cnt + step, L)
                probe = plsc.load_gather(c_v, [cand - 1])
                cnt = jnp.where(probe <= tv, cand, cnt)
                step //= 2
            cnt_v[pl.ds(g * 16, 16)] = cnt

        @plsc.parallel_loop(0, NV, unroll=2)
        def _resolve(g):
            ii = g * 16 + lane
            k = cnt_v[pl.ds(g * 16, 16)]
            a = plsc.load_gather(c_v, [jnp.maximum(k - 1, 0)])
            bv = plsc.load_gather(c_v, [jnp.minimum(k, L - 1)])
            fa = plsc.load_gather(cnt_v, [jnp.maximum(a - 1, 0)])
            fa = jnp.where(a == 0, 0, fa)
            take_a = (k == L) | ((ii - a) <= (bv - ii))
            idx_v[pl.ds(g * 16, 16)] = jnp.where(
                k == 0, 0, jnp.where(take_a, fa, k))

        bufs = ((rvA, ovA, isA, osA), (rvB, ovB, isB, osB))

        def outer(g, carry):
            for b, (rv, ov, isem, osem) in enumerate(bufs):
                q = base + 2 * g + b
                wait_in(rv, isem)

                @pl.when(2 * g + b >= 2)
                def _drain():
                    wait_out(ov, osem)

                @plsc.parallel_loop(0, NV, unroll=8)
                def _chunk(k):
                    kk = k * 16
                    iv = idx_v[pl.ds(kk, 16)]
                    for r in range(_QR):
                        src = [iv + r * L] if r else [iv]
                        ov[pl.ds(r * T + kk, 16)] = plsc.load_gather(rv, src)

                start_out(q, ov, osem)
                nxt = jnp.minimum(q + 2, base + quads_per - 1)
                start_in(nxt, rv, isem)
            return carry

        lax.fori_loop(0, quads_per // 2, outer, 0)
        for rv, ov, isem, osem in bufs:
            wait_in(rv, isem)
            wait_out(ov, osem)

    return samp


_TT = 2048


def _group_masks(P, cg, dtype=jnp.float32):
    g = lax.broadcasted_iota(jnp.int32, (GROUPS, P), 0)
    c = lax.broadcasted_iota(jnp.int32, (GROUPS, P), 1)
    m = (c // cg == g).astype(dtype)
    gT = lax.broadcasted_iota(jnp.int32, (P, GROUPS), 1)
    cT = lax.broadcasted_iota(jnp.int32, (P, GROUPS), 0)
    mT = (cT // cg == gT).astype(dtype)
    return m, mT


def _stats_kernel(T, x_ref, w1_ref, b1_ref, w2_ref, b2_ref, stats_ref, gx, sx):
    t = pl.program_id(1)
    nt = pl.num_programs(1)
    x = x_ref[0]

    @pl.when(t == 0)
    def _init():
        gx[...] = jnp.zeros_like(gx)
        sx[...] = jnp.zeros_like(sx)

    xb = x.astype(jnp.bfloat16)
    gx[...] += lax.dot_general(xb, xb, (((1,), (1,)), ((), ())),
                               preferred_element_type=jnp.float32)
    sx[...] += jnp.sum(x, axis=1, keepdims=True)

    @pl.when(t == nt - 1)
    def _finish():
        Gx = gx[...]
        sxv = sx[...]

        def layer_stats(W, b, cg):
            P = W.shape[0]
            WG = jnp.dot(W, Gx, preferred_element_type=jnp.float32)
            q = jnp.sum(WG * W, axis=1, keepdims=True)
            u = jnp.dot(W, sxv, preferred_element_type=jnp.float32)
            sum_y = u + T * b
            sum_y2 = q + 2.0 * b * u + T * (b * b)
            mask, _ = _group_masks(P, cg)
            Sg = jnp.dot(mask, sum_y, preferred_element_type=jnp.float32)
            Qg = jnp.dot(mask, sum_y2, preferred_element_type=jnp.float32)
            n = cg * T
            mean = Sg / n
            var = Qg / n - mean * mean
            return mean, lax.rsqrt(var + EPS)

        m1, i1 = layer_stats(w1_ref[...], b1_ref[...], 256 // GROUPS)
        m2, i2 = layer_stats(w2_ref[...], b2_ref[...], 512 // GROUPS)
        stats_ref[0, 0:32] = m1
        stats_ref[0, 32:64] = i1
        stats_ref[0, 64:96] = m2
        stats_ref[0, 96:128] = i2


def _blockb_kernel(x_ref, smp_ref, stats_ref, w1_ref, b1_ref, g1_ref, be1_ref,
                   w2_ref, b2_ref, g2_ref, be2_ref, w3_ref, b3_ref,
                   feat_ref, y3_ref, s3_ref):
    t = pl.program_id(1)
    x = x_ref[0]
    smp = smp_ref[0]
    st = stats_ref[0]

    _, mT8 = _group_masks(256, 8)
    _, mT16 = _group_masks(512, 16)
    mask8, _ = _group_masks(256, 8)

    y1 = jnp.dot(w1_ref[...], x, preferred_element_type=jnp.float32) + b1_ref[...]
    m1e = jnp.dot(mT8, st[0:32], preferred_element_type=jnp.float32)
    i1e = jnp.dot(mT8, st[32:64], preferred_element_type=jnp.float32)
    h1 = jnp.maximum((y1 - m1e) * i1e * g1_ref[...] + be1_ref[...], 0.0)

    y2 = jnp.dot(w2_ref[...], x, preferred_element_type=jnp.float32) + b2_ref[...]
    m2e = jnp.dot(mT16, st[64:96], preferred_element_type=jnp.float32)
    i2e = jnp.dot(mT16, st[96:128], preferred_element_type=jnp.float32)
    h2 = jnp.maximum((y2 - m2e) * i2e * g2_ref[...] + be2_ref[...], 0.0)
    feat_ref[0] = h2

    w3 = w3_ref[...].astype(jnp.bfloat16)
    y3 = (jnp.dot(w3[:, 0:256], smp.astype(jnp.bfloat16),
                  preferred_element_type=jnp.float32)
          + jnp.dot(w3[:, 256:768], h2.astype(jnp.bfloat16),
                    preferred_element_type=jnp.float32)
          + jnp.dot(w3[:, 768:1024], h1.astype(jnp.bfloat16),
                    preferred_element_type=jnp.float32)
          + b3_ref[...])
    y3_ref[0] = y3.astype(jnp.bfloat16)

    ssum = jnp.dot(mask8, jnp.sum(y3, axis=1, keepdims=True),
                   preferred_element_type=jnp.float32)
    ssq = jnp.dot(mask8, jnp.sum(y3 * y3, axis=1, keepdims=True),
                  preferred_element_type=jnp.float32)

    @pl.when(t == 0)
    def _init():
        s3_ref[...] = jnp.zeros_like(s3_ref)

    s3_ref[0, 0:32] += ssum
    s3_ref[0, 32:64] += ssq


def _normc_kernel(N3, y3_ref, s3_ref, g3_ref, be3_ref, out_ref):
    y = y3_ref[0].astype(jnp.float32)
    st = s3_ref[0]
    m3 = st[0:32] / N3
    v3 = st[32:64] / N3 - m3 * m3
    i3 = lax.rsqrt(v3 + EPS)
    _, mT8 = _group_masks(256, 8)
    me = jnp.dot(mT8, m3, preferred_element_type=jnp.float32)
    ie = jnp.dot(mT8, i3, preferred_element_type=jnp.float32)
    out_ref[0] = jnp.maximum((y - me) * ie * g3_ref[...] + be3_ref[...], 0.0)


def kernel(feature, frame_level_feature, W1, b1, g1, be1, W2, b2, g2, be2,
           W3, b3, g3, be3):
    B, C, T = feature.shape
    L = frame_level_feature.shape[2]
    P1 = W1.shape[0]
    P2 = W2.shape[0]
    nt = T // _TT

    mean_values = jnp.mean(frame_level_feature[0:1], axis=1)[0]
    mean_values = mean_values / jnp.sum(mean_values)
    cdf_values = jnp.cumsum(mean_values)
    cdf_int = (lax.stop_gradient(cdf_values) * T).astype(jnp.int32)
    cdf_int = jnp.minimum(cdf_int, T - 1)

    sampled = _make_sc_sample(B, C, L, T)(frame_level_feature, cdf_int)

    b1c = b1.reshape(P1, 1)
    b2c = b2.reshape(P2, 1)
    b3c = b3.reshape(P1, 1)
    tts = 4096
    stats = pl.pallas_call(
        functools.partial(_stats_kernel, float(T)),
        grid=(B, T // tts),
        in_specs=[
            pl.BlockSpec((1, C, tts), lambda b, t: (b, 0, t)),
            pl.BlockSpec((P1, C), lambda b, t: (0, 0)),
            pl.BlockSpec((P1, 1), lambda b, t: (0, 0)),
            pl.BlockSpec((P2, C), lambda b, t: (0, 0)),
            pl.BlockSpec((P2, 1), lambda b, t: (0, 0)),
        ],
        out_specs=pl.BlockSpec((1, 128, 1), lambda b, t: (b, 0, 0)),
        out_shape=jax.ShapeDtypeStruct((B, 128, 1), jnp.float32),
        scratch_shapes=[
            pltpu.VMEM((C, C), jnp.float32),
            pltpu.VMEM((C, 1), jnp.float32),
        ],
    )(feature, W1, b1c, W2, b2c)

    feat, y3raw, s3 = pl.pallas_call(
        _blockb_kernel,
        grid=(B, nt),
        in_specs=[
            pl.BlockSpec((1, C, _TT), lambda b, t: (b, 0, t)),
            pl.BlockSpec((1, C, _TT), lambda b, t: (b, 0, t)),
            pl.BlockSpec((1, 128, 1), lambda b, t: (b, 0, 0)),
            pl.BlockSpec((P1, C), lambda b, t: (0, 0)),
            pl.BlockSpec((P1, 1), lambda b, t: (0, 0)),
            pl.BlockSpec((P1, 1), lambda b, t: (0, 0)),
            pl.BlockSpec((P1, 1), lambda b, t: (0, 0)),
            pl.BlockSpec((P2, C), lambda b, t: (0, 0)),
            pl.BlockSpec((P2, 1), lambda b, t: (0, 0)),
            pl.BlockSpec((P2, 1), lambda b, t: (0, 0)),
            pl.BlockSpec((P2, 1), lambda b, t: (0, 0)),
            pl.BlockSpec((P1, 4 * P1), lambda b, t: (0, 0)),
            pl.BlockSpec((P1, 1), lambda b, t: (0, 0)),
        ],
        out_specs=[
            pl.BlockSpec((1, P2, _TT), lambda b, t: (b, 0, t)),
            pl.BlockSpec((1, P1, _TT), lambda b, t: (b, 0, t)),
            pl.BlockSpec((1, 64, 1), lambda b, t: (b, 0, 0)),
        ],
        out_shape=[
            jax.ShapeDtypeStruct((B, P2, T), jnp.float32),
            jax.ShapeDtypeStruct((B, P1, T), jnp.bfloat16),
            jax.ShapeDtypeStruct((B, 64, 1), jnp.float32),
        ],
    )(feature, sampled, stats, W1, b1c, g1.reshape(P1, 1), be1.reshape(P1, 1),
      W2, b2c, g2.reshape(P2, 1), be2.reshape(P2, 1), W3, b3c)

    mixed = pl.pallas_call(
        functools.partial(_normc_kernel, float((P1 // GROUPS) * T)),
        grid=(B, nt),
        in_specs=[
            pl.BlockSpec((1, P1, _TT), lambda b, t: (b, 0, t)),
            pl.BlockSpec((1, 64, 1), lambda b, t: (b, 0, 0)),
            pl.BlockSpec((P1, 1), lambda b, t: (0, 0)),
            pl.BlockSpec((P1, 1), lambda b, t: (0, 0)),
        ],
        out_specs=pl.BlockSpec((1, P1, _TT), lambda b, t: (b, 0, t)),
        out_shape=jax.ShapeDtypeStruct((B, P1, T), jnp.float32),
    )(y3raw, s3, g3.reshape(P1, 1), be3.reshape(P1, 1))

    return (mixed, feat)
